# baseline (device time: 17610 ns/iter reference)
import jax
import jax.numpy as jnp
from jax import lax
from jax.experimental import pallas as pl
from jax.experimental.pallas import tpu as pltpu

N_DEV = 4
B, SQ, SKV, D_MODEL, DH = 2, 256, 256, 512, 64
H_LOC = 4
QR = (B * SQ) // N_DEV


def kernel(x, Wq, K_ext, V_ext, Wo):
    my_i = lax.axis_index("i")
    K_loc = lax.dynamic_slice_in_dim(K_ext, my_i * H_LOC, H_LOC, axis=2)
    V_loc = lax.dynamic_slice_in_dim(V_ext, my_i * H_LOC, H_LOC, axis=2)
    K_loc = jnp.transpose(K_loc.astype(jnp.bfloat16), (0, 2, 1, 3))
    V_loc = jnp.transpose(V_loc.astype(jnp.bfloat16), (0, 2, 1, 3))

    def body(x_ref, wq_ref, k_ref, v_ref, wo_ref, out_ref,
             rs_send, rs_recv, ag_send, ag_recv, acc_ref,
             rs_send_sems, rs_recv_sems, ag_send_sems, ag_recv_sems):
        me = lax.axis_index("i")

        barrier_sem = pltpu.get_barrier_semaphore()
        for d in range(1, N_DEV):
            pl.semaphore_signal(
                barrier_sem, inc=1,
                device_id=(lax.rem(me + d, N_DEV),),
                device_id_type=pl.DeviceIdType.MESH,
            )
        pl.semaphore_wait(barrier_sem, N_DEV - 1)

        wq = wq_ref[...].astype(jnp.bfloat16)
        wo = wo_ref[...].astype(jnp.bfloat16)

        qb = lax.broadcasted_iota(jnp.int32, (SQ, SKV), 0) // 64
        kb = lax.broadcasted_iota(jnp.int32, (SQ, SKV), 1) // 64
        keep = (qb == kb) | ((kb % 4) == (qb % 4))
        bias = jnp.where(keep, 0.0, -1e9).astype(jnp.float32)

        x2 = x_ref[...].astype(jnp.bfloat16).reshape(B * SQ, D_MODEL)
        q_all = jnp.dot(x2, wq, preferred_element_type=jnp.float32)
        q_all = (q_all * 0.125).astype(jnp.bfloat16)

        ctx_b = []
        for b in range(B):
            ctx_h = []
            for h in range(H_LOC):
                qh = q_all[b * SQ:(b + 1) * SQ, h * DH:(h + 1) * DH]
                kh = k_ref[b, h]
                vh = v_ref[b, h]
                s = lax.dot_general(
                    qh, kh, (((1,), (1,)), ((), ())),
                    preferred_element_type=jnp.float32)
                w = jnp.exp(s + bias)
                denom = jnp.sum(w, axis=-1, keepdims=True)
                ctx_un = jnp.dot(w.astype(jnp.bfloat16), vh,
                                 preferred_element_type=jnp.float32)
                ctx_h.append(ctx_un / denom)
            ctx_b.append(jnp.concatenate(ctx_h, axis=-1).astype(jnp.bfloat16))
        ctx = jnp.concatenate(ctx_b, axis=0)

        rs_rdmas = []
        for q in range(N_DEV):
            partial_q = jnp.dot(ctx[q * QR:(q + 1) * QR, :], wo,
                                preferred_element_type=jnp.float32)
            rs_send[q] = partial_q.astype(jnp.bfloat16)

            @pl.when(q == me)
            def _(pq=partial_q):
                acc_ref[...] = pq

            rdma = pltpu.make_async_remote_copy(
                src_ref=rs_send.at[q],
                dst_ref=rs_recv.at[me],
                send_sem=rs_send_sems.at[q],
                recv_sem=rs_recv_sems.at[me],
                device_id=(q,),
                device_id_type=pl.DeviceIdType.MESH,
            )

            @pl.when(q != me)
            def _(r=rdma):
                r.start()

            rs_rdmas.append((rdma, q))

        for m in range(N_DEV):
            recv = pltpu.make_async_remote_copy(
                src_ref=rs_send.at[m],
                dst_ref=rs_recv.at[m],
                send_sem=rs_send_sems.at[m],
                recv_sem=rs_recv_sems.at[m],
                device_id=(me,),
                device_id_type=pl.DeviceIdType.MESH,
            )

            @pl.when(m != me)
            def _(r=recv):
                r.wait_recv()
                acc_ref[...] += rs_recv[m].astype(jnp.float32)

        ag_send[...] = acc_ref[...].astype(jnp.bfloat16)
        ag_rdmas = []
        for d in range(1, N_DEV):
            rdma = pltpu.make_async_remote_copy(
                src_ref=ag_send,
                dst_ref=ag_recv.at[me],
                send_sem=ag_send_sems.at[d - 1],
                recv_sem=ag_recv_sems.at[me],
                device_id=(lax.rem(me + d, N_DEV),),
                device_id_type=pl.DeviceIdType.MESH,
            )
            rdma.start()
            ag_rdmas.append(rdma)

        for q in range(N_DEV):
            b, r0 = divmod(q * QR, SQ)

            @pl.when(q == me)
            def _(b=b, r0=r0):
                out_ref[b, r0:r0 + QR, :] = acc_ref[...]

            recv = pltpu.make_async_remote_copy(
                src_ref=ag_send,
                dst_ref=ag_recv.at[q],
                send_sem=ag_send_sems.at[0],
                recv_sem=ag_recv_sems.at[q],
                device_id=(me,),
                device_id_type=pl.DeviceIdType.MESH,
            )

            @pl.when(q != me)
            def _(r=recv, b=b, r0=r0, q=q):
                r.wait_recv()
                out_ref[b, r0:r0 + QR, :] = ag_recv[q].astype(jnp.float32)

        for rdma, q in rs_rdmas:
            @pl.when(q != me)
            def _(r=rdma):
                r.wait_send()
        for rdma in ag_rdmas:
            rdma.wait_send()

    return pl.pallas_call(
        body,
        out_shape=jax.ShapeDtypeStruct((B, SQ, D_MODEL), jnp.float32),
        in_specs=[pl.BlockSpec(memory_space=pltpu.VMEM)] * 5,
        out_specs=pl.BlockSpec(memory_space=pltpu.VMEM),
        scratch_shapes=[
            pltpu.VMEM((N_DEV, QR, D_MODEL), jnp.bfloat16),
            pltpu.VMEM((N_DEV, QR, D_MODEL), jnp.bfloat16),
            pltpu.VMEM((QR, D_MODEL), jnp.bfloat16),
            pltpu.VMEM((N_DEV, QR, D_MODEL), jnp.bfloat16),
            pltpu.VMEM((QR, D_MODEL), jnp.float32),
            pltpu.SemaphoreType.DMA((N_DEV,)),
            pltpu.SemaphoreType.DMA((N_DEV,)),
            pltpu.SemaphoreType.DMA((N_DEV - 1,)),
            pltpu.SemaphoreType.DMA((N_DEV,)),
        ],
        compiler_params=pltpu.CompilerParams(collective_id=0),
    )(x, Wq, K_loc, V_loc, Wo)


# device time: 17327 ns/iter; 1.0163x vs baseline; 1.0163x over previous
import jax
import jax.numpy as jnp
from jax import lax
from jax.experimental import pallas as pl
from jax.experimental.pallas import tpu as pltpu

N_DEV = 4
B, SQ, SKV, D_MODEL, DH = 2, 256, 256, 512, 64
H_LOC = 4
QR = (B * SQ) // N_DEV


def kernel(x, Wq, K_ext, V_ext, Wo):
    my_i = lax.axis_index("i")
    K_loc = lax.dynamic_slice_in_dim(K_ext, my_i * H_LOC, H_LOC, axis=2)
    V_loc = lax.dynamic_slice_in_dim(V_ext, my_i * H_LOC, H_LOC, axis=2)
    K_loc = K_loc.astype(jnp.bfloat16).reshape(B, SKV, H_LOC * DH)
    V_loc = V_loc.astype(jnp.bfloat16).reshape(B, SKV, H_LOC * DH)

    def body(x_ref, wq_ref, k_ref, v_ref, wo_ref, out_ref,
             rs_send, rs_recv, ag_send, ag_recv, acc_ref,
             rs_send_sems, rs_recv_sems, ag_send_sems, ag_recv_sems):
        me = lax.axis_index("i")

        barrier_sem = pltpu.get_barrier_semaphore()
        for d in range(1, N_DEV):
            pl.semaphore_signal(
                barrier_sem, inc=1,
                device_id=(lax.rem(me + d, N_DEV),),
                device_id_type=pl.DeviceIdType.MESH,
            )
        pl.semaphore_wait(barrier_sem, N_DEV - 1)

        wq = wq_ref[...].astype(jnp.bfloat16)
        wo = wo_ref[...].astype(jnp.bfloat16)

        qb = lax.broadcasted_iota(jnp.int32, (SQ, SKV), 0) // 64
        kb = lax.broadcasted_iota(jnp.int32, (SQ, SKV), 1) // 64
        keep = (qb == kb) | ((kb % 4) == (qb % 4))
        bias = jnp.where(keep, 0.0, -1e9).astype(jnp.float32)

        x2 = x_ref[...].astype(jnp.bfloat16).reshape(B * SQ, D_MODEL)
        q_all = jnp.dot(x2, wq, preferred_element_type=jnp.float32)
        q_all = (q_all * 0.125).astype(jnp.bfloat16)

        ctx_b = []
        for b in range(B):
            ctx_h = []
            for h in range(H_LOC):
                qh = q_all[b * SQ:(b + 1) * SQ, h * DH:(h + 1) * DH]
                kh = k_ref[b, :, h * DH:(h + 1) * DH]
                vh = v_ref[b, :, h * DH:(h + 1) * DH]
                s = lax.dot_general(
                    qh, kh, (((1,), (1,)), ((), ())),
                    preferred_element_type=jnp.float32)
                w = jnp.exp(s + bias)
                denom = jnp.sum(w, axis=-1, keepdims=True)
                ctx_un = jnp.dot(w.astype(jnp.bfloat16), vh,
                                 preferred_element_type=jnp.float32)
                ctx_h.append(ctx_un / denom)
            ctx_b.append(jnp.concatenate(ctx_h, axis=-1).astype(jnp.bfloat16))
        ctx = jnp.concatenate(ctx_b, axis=0)

        rs_rdmas = []
        for q in range(N_DEV):
            partial_q = jnp.dot(ctx[q * QR:(q + 1) * QR, :], wo,
                                preferred_element_type=jnp.float32)
            rs_send[q] = partial_q.astype(jnp.bfloat16)

            @pl.when(q == me)
            def _(pq=partial_q):
                acc_ref[...] = pq

            rdma = pltpu.make_async_remote_copy(
                src_ref=rs_send.at[q],
                dst_ref=rs_recv.at[me],
                send_sem=rs_send_sems.at[q],
                recv_sem=rs_recv_sems.at[me],
                device_id=(q,),
                device_id_type=pl.DeviceIdType.MESH,
            )

            @pl.when(q != me)
            def _(r=rdma):
                r.start()

            rs_rdmas.append((rdma, q))

        for m in range(N_DEV):
            recv = pltpu.make_async_remote_copy(
                src_ref=rs_send.at[m],
                dst_ref=rs_recv.at[m],
                send_sem=rs_send_sems.at[m],
                recv_sem=rs_recv_sems.at[m],
                device_id=(me,),
                device_id_type=pl.DeviceIdType.MESH,
            )

            @pl.when(m != me)
            def _(r=recv):
                r.wait_recv()
                acc_ref[...] += rs_recv[m].astype(jnp.float32)

        ag_send[...] = acc_ref[...].astype(jnp.bfloat16)
        ag_rdmas = []
        for d in range(1, N_DEV):
            rdma = pltpu.make_async_remote_copy(
                src_ref=ag_send,
                dst_ref=ag_recv.at[me],
                send_sem=ag_send_sems.at[d - 1],
                recv_sem=ag_recv_sems.at[me],
                device_id=(lax.rem(me + d, N_DEV),),
                device_id_type=pl.DeviceIdType.MESH,
            )
            rdma.start()
            ag_rdmas.append(rdma)

        for q in range(N_DEV):
            b, r0 = divmod(q * QR, SQ)

            @pl.when(q == me)
            def _(b=b, r0=r0):
                out_ref[b, r0:r0 + QR, :] = acc_ref[...]

            recv = pltpu.make_async_remote_copy(
                src_ref=ag_send,
                dst_ref=ag_recv.at[q],
                send_sem=ag_send_sems.at[0],
                recv_sem=ag_recv_sems.at[q],
                device_id=(me,),
                device_id_type=pl.DeviceIdType.MESH,
            )

            @pl.when(q != me)
            def _(r=recv, b=b, r0=r0, q=q):
                r.wait_recv()
                out_ref[b, r0:r0 + QR, :] = ag_recv[q].astype(jnp.float32)

        for rdma, q in rs_rdmas:
            @pl.when(q != me)
            def _(r=rdma):
                r.wait_send()
        for rdma in ag_rdmas:
            rdma.wait_send()

    return pl.pallas_call(
        body,
        out_shape=jax.ShapeDtypeStruct((B, SQ, D_MODEL), jnp.float32),
        in_specs=[pl.BlockSpec(memory_space=pltpu.VMEM)] * 5,
        out_specs=pl.BlockSpec(memory_space=pltpu.VMEM),
        scratch_shapes=[
            pltpu.VMEM((N_DEV, QR, D_MODEL), jnp.bfloat16),
            pltpu.VMEM((N_DEV, QR, D_MODEL), jnp.bfloat16),
            pltpu.VMEM((QR, D_MODEL), jnp.bfloat16),
            pltpu.VMEM((N_DEV, QR, D_MODEL), jnp.bfloat16),
            pltpu.VMEM((QR, D_MODEL), jnp.float32),
            pltpu.SemaphoreType.DMA((N_DEV,)),
            pltpu.SemaphoreType.DMA((N_DEV,)),
            pltpu.SemaphoreType.DMA((N_DEV - 1,)),
            pltpu.SemaphoreType.DMA((N_DEV,)),
        ],
        compiler_params=pltpu.CompilerParams(collective_id=0),
    )(x, Wq, K_loc, V_loc, Wo)


# device time: 17102 ns/iter; 1.0297x vs baseline; 1.0132x over previous
import jax
import jax.numpy as jnp
from jax import lax
from jax.experimental import pallas as pl
from jax.experimental.pallas import tpu as pltpu

N_DEV = 4
B, SQ, SKV, D_MODEL, DH = 2, 256, 256, 512, 64
H_LOC = 4
QR = (B * SQ) // N_DEV


def kernel(x, Wq, K_ext, V_ext, Wo):
    my_i = lax.axis_index("i")
    K_loc = lax.dynamic_slice_in_dim(K_ext, my_i * H_LOC, H_LOC, axis=2)
    V_loc = lax.dynamic_slice_in_dim(V_ext, my_i * H_LOC, H_LOC, axis=2)
    K_loc = K_loc.astype(jnp.bfloat16).reshape(B, SKV, H_LOC * DH)
    V_loc = V_loc.astype(jnp.bfloat16).reshape(B, SKV, H_LOC * DH)

    def body(x_ref, wq_ref, k_ref, v_ref, wo_ref, out_ref,
             rs_send, rs_recv, ag_send, ag_recv, acc_ref, q_scr,
             rs_send_sems, rs_recv_sems, ag_send_sems, ag_recv_sems):
        me = lax.axis_index("i")

        barrier_sem = pltpu.get_barrier_semaphore()
        for d in range(1, N_DEV):
            pl.semaphore_signal(
                barrier_sem, inc=1,
                device_id=(lax.rem(me + d, N_DEV),),
                device_id_type=pl.DeviceIdType.MESH,
            )
        pl.semaphore_wait(barrier_sem, N_DEV - 1)

        wq = wq_ref[...].astype(jnp.bfloat16)
        wo = wo_ref[...].astype(jnp.bfloat16)

        qb = lax.broadcasted_iota(jnp.int32, (SQ, SKV), 0) // 64
        kb = lax.broadcasted_iota(jnp.int32, (SQ, SKV), 1) // 64
        keep = (qb == kb) | ((kb % 4) == (qb % 4))
        bias_full = jnp.where(keep, 0.0, -1e9).astype(jnp.float32)
        bias_lo = bias_full[:QR, :]
        bias_hi = bias_full[QR:, :]

        x2 = x_ref[...].astype(jnp.bfloat16).reshape(B * SQ, D_MODEL)
        q_all = jnp.dot(x2, wq, preferred_element_type=jnp.float32)
        q_scr[...] = (q_all * 0.125).astype(jnp.bfloat16)

        def quarter_partial(qi):
            bq = qi // 2
            hi = (qi % 2) == 1
            bias = jnp.where(hi, bias_hi, bias_lo)
            ctx_h = []
            for h in range(H_LOC):
                qh = q_scr[pl.ds(qi * QR, QR), h * DH:(h + 1) * DH]
                kh = k_ref[bq, :, h * DH:(h + 1) * DH]
                vh = v_ref[bq, :, h * DH:(h + 1) * DH]
                s = lax.dot_general(
                    qh, kh, (((1,), (1,)), ((), ())),
                    preferred_element_type=jnp.float32)
                w = jnp.exp(s + bias)
                denom = jnp.sum(w, axis=-1, keepdims=True)
                ctx_un = jnp.dot(w.astype(jnp.bfloat16), vh,
                                 preferred_element_type=jnp.float32)
                ctx_h.append(ctx_un / denom)
            ctxq = jnp.concatenate(ctx_h, axis=-1).astype(jnp.bfloat16)
            return jnp.dot(ctxq, wo, preferred_element_type=jnp.float32)

        rs_rdmas = []
        for dq in range(1, N_DEV):
            qi = lax.rem(me + dq, N_DEV)
            rs_send[dq - 1] = quarter_partial(qi).astype(jnp.bfloat16)
            rdma = pltpu.make_async_remote_copy(
                src_ref=rs_send.at[dq - 1],
                dst_ref=rs_recv.at[me],
                send_sem=rs_send_sems.at[dq - 1],
                recv_sem=rs_recv_sems.at[me],
                device_id=(qi,),
                device_id_type=pl.DeviceIdType.MESH,
            )
            rdma.start()
            rs_rdmas.append(rdma)

        acc_ref[...] = quarter_partial(me)

        for m in range(N_DEV):
            recv = pltpu.make_async_remote_copy(
                src_ref=rs_send.at[0],
                dst_ref=rs_recv.at[m],
                send_sem=rs_send_sems.at[0],
                recv_sem=rs_recv_sems.at[m],
                device_id=(me,),
                device_id_type=pl.DeviceIdType.MESH,
            )

            @pl.when(m != me)
            def _(r=recv):
                r.wait_recv()
                acc_ref[...] += rs_recv[m].astype(jnp.float32)

        ag_send[...] = acc_ref[...].astype(jnp.bfloat16)
        ag_rdmas = []
        for d in range(1, N_DEV):
            rdma = pltpu.make_async_remote_copy(
                src_ref=ag_send,
                dst_ref=ag_recv.at[me],
                send_sem=ag_send_sems.at[d - 1],
                recv_sem=ag_recv_sems.at[me],
                device_id=(lax.rem(me + d, N_DEV),),
                device_id_type=pl.DeviceIdType.MESH,
            )
            rdma.start()
            ag_rdmas.append(rdma)

        for q in range(N_DEV):
            b, r0 = divmod(q * QR, SQ)

            @pl.when(q == me)
            def _(b=b, r0=r0):
                out_ref[b, r0:r0 + QR, :] = acc_ref[...]

            recv = pltpu.make_async_remote_copy(
                src_ref=ag_send,
                dst_ref=ag_recv.at[q],
                send_sem=ag_send_sems.at[0],
                recv_sem=ag_recv_sems.at[q],
                device_id=(me,),
                device_id_type=pl.DeviceIdType.MESH,
            )

            @pl.when(q != me)
            def _(r=recv, b=b, r0=r0, q=q):
                r.wait_recv()
                out_ref[b, r0:r0 + QR, :] = ag_recv[q].astype(jnp.float32)

        for rdma in rs_rdmas:
            rdma.wait_send()
        for rdma in ag_rdmas:
            rdma.wait_send()

    return pl.pallas_call(
        body,
        out_shape=jax.ShapeDtypeStruct((B, SQ, D_MODEL), jnp.float32),
        in_specs=[pl.BlockSpec(memory_space=pltpu.VMEM)] * 5,
        out_specs=pl.BlockSpec(memory_space=pltpu.VMEM),
        scratch_shapes=[
            pltpu.VMEM((N_DEV - 1, QR, D_MODEL), jnp.bfloat16),
            pltpu.VMEM((N_DEV, QR, D_MODEL), jnp.bfloat16),
            pltpu.VMEM((QR, D_MODEL), jnp.bfloat16),
            pltpu.VMEM((N_DEV, QR, D_MODEL), jnp.bfloat16),
            pltpu.VMEM((QR, D_MODEL), jnp.float32),
            pltpu.VMEM((B * SQ, H_LOC * DH), jnp.bfloat16),
            pltpu.SemaphoreType.DMA((N_DEV - 1,)),
            pltpu.SemaphoreType.DMA((N_DEV,)),
            pltpu.SemaphoreType.DMA((N_DEV - 1,)),
            pltpu.SemaphoreType.DMA((N_DEV,)),
        ],
        compiler_params=pltpu.CompilerParams(collective_id=0),
    )(x, Wq, K_loc, V_loc, Wo)


# device time: 16896 ns/iter; 1.0423x vs baseline; 1.0122x over previous
import jax
import jax.numpy as jnp
from jax import lax
from jax.experimental import pallas as pl
from jax.experimental.pallas import tpu as pltpu

N_DEV = 4
B, SQ, SKV, D_MODEL, DH = 2, 256, 256, 512, 64
H_LOC = 4
QR = (B * SQ) // N_DEV


def kernel(x, Wq, K_ext, V_ext, Wo):
    my_i = lax.axis_index("i")
    K_loc = lax.dynamic_slice_in_dim(K_ext, my_i * H_LOC, H_LOC, axis=2)
    V_loc = lax.dynamic_slice_in_dim(V_ext, my_i * H_LOC, H_LOC, axis=2)
    K_loc = K_loc.astype(jnp.bfloat16).reshape(B, SKV, H_LOC * DH)
    V_loc = V_loc.astype(jnp.bfloat16).reshape(B, SKV, H_LOC * DH)

    def body(x_ref, wq_ref, k_ref, v_ref, wo_ref, out_ref,
             rs_send, rs_recv, ag_send, ag_recv, acc_ref, q_scr,
             rs_send_sems, rs_recv_sems, ag_send_sems, ag_recv_sems):
        me = lax.axis_index("i")

        barrier_sem = pltpu.get_barrier_semaphore()
        for d in range(1, N_DEV):
            pl.semaphore_signal(
                barrier_sem, inc=1,
                device_id=(lax.rem(me + d, N_DEV),),
                device_id_type=pl.DeviceIdType.MESH,
            )
        pl.semaphore_wait(barrier_sem, N_DEV - 1)

        wq = wq_ref[...].astype(jnp.bfloat16)
        wo = wo_ref[...].astype(jnp.bfloat16)

        qb = lax.broadcasted_iota(jnp.int32, (SQ, SKV), 0) // 64
        kb = lax.broadcasted_iota(jnp.int32, (SQ, SKV), 1) // 64
        keep = (qb == kb) | ((kb % 4) == (qb % 4))
        bias_full = jnp.where(keep, 0.0, -1e9).astype(jnp.float32)
        bias_lo = bias_full[:QR, :]
        bias_hi = bias_full[QR:, :]

        x2 = x_ref[...].astype(jnp.bfloat16).reshape(B * SQ, D_MODEL)
        q_all = jnp.dot(x2, wq, preferred_element_type=jnp.float32)
        q_scr[...] = (q_all * 0.125).astype(jnp.bfloat16)

        def quarter_partial(qi):
            bq = qi // 2
            hi = (qi % 2) == 1
            bias = jnp.where(hi, bias_hi, bias_lo)
            ctx_h = []
            for h in range(H_LOC):
                qh = q_scr[pl.ds(qi * QR, QR), h * DH:(h + 1) * DH]
                kh = k_ref[bq, :, h * DH:(h + 1) * DH]
                vh = v_ref[bq, :, h * DH:(h + 1) * DH]
                s = lax.dot_general(
                    qh, kh, (((1,), (1,)), ((), ())),
                    preferred_element_type=jnp.float32)
                w = jnp.exp(s + bias)
                denom = jnp.sum(w, axis=-1, keepdims=True)
                ctx_un = jnp.dot(w.astype(jnp.bfloat16), vh,
                                 preferred_element_type=jnp.float32)
                ctx_h.append(ctx_un / denom)
            ctxq = jnp.concatenate(ctx_h, axis=-1).astype(jnp.bfloat16)
            return jnp.dot(ctxq, wo, preferred_element_type=jnp.float32)

        rs_rdmas = []
        for dq in range(1, N_DEV):
            qi = lax.rem(me + dq, N_DEV)
            rs_send[dq - 1] = quarter_partial(qi).astype(jnp.bfloat16)
            rdma = pltpu.make_async_remote_copy(
                src_ref=rs_send.at[dq - 1],
                dst_ref=rs_recv.at[me],
                send_sem=rs_send_sems.at[dq - 1],
                recv_sem=rs_recv_sems.at[me],
                device_id=(qi,),
                device_id_type=pl.DeviceIdType.MESH,
            )
            rdma.start()
            rs_rdmas.append(rdma)

        acc_ref[...] = quarter_partial(me)

        for m in range(N_DEV):
            recv = pltpu.make_async_remote_copy(
                src_ref=rs_send.at[0],
                dst_ref=rs_recv.at[m],
                send_sem=rs_send_sems.at[0],
                recv_sem=rs_recv_sems.at[m],
                device_id=(me,),
                device_id_type=pl.DeviceIdType.MESH,
            )

            @pl.when(m != me)
            def _(r=recv):
                r.wait_recv()
                acc_ref[...] += rs_recv[m].astype(jnp.float32)

        ag_send[...] = acc_ref[...].astype(jnp.bfloat16)
        ag_rdmas = []
        for d in range(1, N_DEV):
            rdma = pltpu.make_async_remote_copy(
                src_ref=ag_send,
                dst_ref=ag_recv.at[me],
                send_sem=ag_send_sems.at[d - 1],
                recv_sem=ag_recv_sems.at[me],
                device_id=(lax.rem(me + d, N_DEV),),
                device_id_type=pl.DeviceIdType.MESH,
            )
            rdma.start()
            ag_rdmas.append(rdma)

        for q in range(N_DEV):
            b, r0 = divmod(q * QR, SQ)

            @pl.when(q == me)
            def _(b=b, r0=r0):
                out_ref[b, r0:r0 + QR, :] = acc_ref[...].astype(jnp.bfloat16)

            recv = pltpu.make_async_remote_copy(
                src_ref=ag_send,
                dst_ref=ag_recv.at[q],
                send_sem=ag_send_sems.at[0],
                recv_sem=ag_recv_sems.at[q],
                device_id=(me,),
                device_id_type=pl.DeviceIdType.MESH,
            )

            @pl.when(q != me)
            def _(r=recv, b=b, r0=r0, q=q):
                r.wait_recv()
                out_ref[b, r0:r0 + QR, :] = ag_recv[q]

        for rdma in rs_rdmas:
            rdma.wait_send()
        for rdma in ag_rdmas:
            rdma.wait_send()

    return pl.pallas_call(
        body,
        out_shape=jax.ShapeDtypeStruct((B, SQ, D_MODEL), jnp.bfloat16),
        in_specs=[pl.BlockSpec(memory_space=pltpu.VMEM)] * 5,
        out_specs=pl.BlockSpec(memory_space=pltpu.VMEM),
        scratch_shapes=[
            pltpu.VMEM((N_DEV - 1, QR, D_MODEL), jnp.bfloat16),
            pltpu.VMEM((N_DEV, QR, D_MODEL), jnp.bfloat16),
            pltpu.VMEM((QR, D_MODEL), jnp.bfloat16),
            pltpu.VMEM((N_DEV, QR, D_MODEL), jnp.bfloat16),
            pltpu.VMEM((QR, D_MODEL), jnp.float32),
            pltpu.VMEM((B * SQ, H_LOC * DH), jnp.bfloat16),
            pltpu.SemaphoreType.DMA((N_DEV - 1,)),
            pltpu.SemaphoreType.DMA((N_DEV,)),
            pltpu.SemaphoreType.DMA((N_DEV - 1,)),
            pltpu.SemaphoreType.DMA((N_DEV,)),
        ],
        compiler_params=pltpu.CompilerParams(collective_id=0),
    )(x, Wq, K_loc, V_loc, Wo)
